# baseline (speedup 1.0000x reference)
import functools

import jax
import jax.numpy as jnp
from jax import lax
from jax.experimental import pallas as pl
from jax.experimental.pallas import tpu as pltpu
from jax.experimental.pallas import tpu_sc as plsc

D = 128
TAIL = 16
DP = D + TAIL
PW = D // 2
TW = PW + TAIL
LANES = 16
BN = 1000
B = 48


def _norm_body(x_ref, out_ref):
    x = x_ref[...]
    nrm = jnp.sqrt(jnp.sum(x * x, axis=1, keepdims=True))
    xn = x / (nrm + 1e-12)
    u = jax.lax.bitcast_convert_type(xn.astype(jnp.bfloat16),
                                     jnp.uint16).astype(jnp.uint32)
    groups = []
    for k in range(4):
        lo = u[:, 32 * k:32 * k + 16]
        hi = u[:, 32 * k + 16:32 * k + 32]
        groups.append((hi << 16) | lo)
    packed = jax.lax.bitcast_convert_type(
        jnp.concatenate(groups, axis=1), jnp.float32)
    tail = jnp.broadcast_to(nrm, (x.shape[0], TAIL))
    out_ref[...] = jnp.concatenate([packed, tail], axis=1)


def _combine_body(a0_ref, a1_ref, out_ref):
    s = a0_ref[...] + a1_ref[...]
    out_ref[...] = s[:, :D] / (s[:, D:D + 1] + 1e-16)


@functools.lru_cache(maxsize=None)
def _make_sc(N, EPWP):
    info = plsc.get_sparse_core_info()
    NC, NS = info.num_cores, info.num_subcores
    NCHUNK = EPWP // B
    RPT = N // NS
    NZ = RPT // B
    REM = RPT - NZ * B
    mesh = plsc.VectorSubcoreMesh(core_axis_name="c", subcore_axis_name="s")

    @functools.partial(
        pl.kernel,
        out_type=jax.ShapeDtypeStruct((NC * N, DP), jnp.float32),
        mesh=mesh,
        compiler_params=pltpu.CompilerParams(use_tc_tiling_on_sc=False,
                                             needs_layout_passes=False),
        scratch_types=[
            pltpu.VMEM((B,), jnp.int32),
            pltpu.VMEM((B,), jnp.int32),
            pltpu.VMEM((B,), jnp.int32),
            pltpu.VMEM((B,), jnp.int32),
            pltpu.VMEM((B, TW), jnp.float32),
            pltpu.VMEM((B, TW), jnp.float32),
            pltpu.VMEM((B, TW), jnp.float32),
            pltpu.VMEM((B, TW), jnp.float32),
            pltpu.VMEM((B, DP), jnp.float32),
            pltpu.VMEM((B, DP), jnp.float32),
            pltpu.VMEM((B,), jnp.int32),
            pltpu.VMEM((B,), jnp.int32),
            pltpu.VMEM((LANES,), jnp.float32),
            pltpu.VMEM_SHARED((N + 1, DP), jnp.float32),
        ] + [pltpu.SemaphoreType.DMA] * 8,
    )
    def sc(table, srcs2, dsts2, beta16, out,
           sidx0, didx0, sidx1, didx1, sr0, dr0, sr1, dr1,
           obuf0, obuf1, sdidx0, sdidx1, bvec, acc,
           gs0, gd0, gs1, gd1, o0, o1, i0, i1):
        c = lax.axis_index("c")
        s = lax.axis_index("s")
        wid = c * NS + s
        zero16 = jnp.zeros((LANES,), jnp.float32)
        idxs = ((sidx0, didx0, i0), (sidx1, didx1, i1))
        rows = ((sr0, dr0, gs0, gd0), (sr1, dr1, gs1, gd1))
        obufs = ((obuf0, o0, sdidx0), (obuf1, o1, sdidx1))

        @pl.loop(0, B)
        def _zero_obuf(r):
            for k in range(DP // LANES):
                obuf0[r, pl.ds(k * LANES, LANES)] = zero16
                obuf1[r, pl.ds(k * LANES, LANES)] = zero16

        lane = lax.iota(jnp.int32, LANES)

        row0 = s * RPT
        for j in range(NZ):
            pltpu.sync_copy(obuf0, acc.at[pl.ds(row0 + j * B, B)])
        if REM:
            pltpu.sync_copy(obuf0.at[pl.ds(0, REM)],
                            acc.at[pl.ds(row0 + NZ * B, REM)])
        pltpu.sync_copy(beta16, bvec)

        def i_issue(p, ci):
            si, di, isem = idxs[p]
            pltpu.async_copy(srcs2.at[wid, pl.ds(ci * B, B)], si, isem)
            pltpu.async_copy(dsts2.at[wid, pl.ds(ci * B, B)], di, isem)

        def i_wait(p, ci):
            si, di, isem = idxs[p]
            pltpu.make_async_copy(srcs2.at[wid, pl.ds(ci * B, B)],
                                  si, isem).wait()
            pltpu.make_async_copy(dsts2.at[wid, pl.ds(ci * B, B)],
                                  di, isem).wait()

        def g_issue(p):
            si, di, _ = idxs[p]
            sr, dr, ss, sd = rows[p]
            pltpu.async_copy(table.at[si], sr, ss)
            pltpu.async_copy(table.at[di], dr, sd)

        def g_wait(p):
            si, di, _ = idxs[p]
            sr, dr, ss, sd = rows[p]
            pltpu.make_async_copy(table.at[si], sr, ss).wait()
            pltpu.make_async_copy(table.at[di], dr, sd).wait()

        def c_pre(p):
            _, di, _ = idxs[p]
            ob, osem, sd = obufs[p]
            pltpu.make_async_copy(ob, acc.at[sd], osem).wait()
            for t in range(B // LANES):
                sd[pl.ds(t * LANES, LANES)] = di[pl.ds(t * LANES, LANES)]

        pltpu.sync_copy(srcs2.at[wid, pl.ds(0, B)], sidx0)
        pltpu.sync_copy(dsts2.at[wid, pl.ds(0, B)], didx0)
        i_issue(1, 1)
        for t in range(B // LANES):
            sdidx0[pl.ds(t * LANES, LANES)] = didx0[pl.ds(t * LANES, LANES)]
            sdidx1[pl.ds(t * LANES, LANES)] = didx0[pl.ds(t * LANES, LANES)]
        pltpu.async_copy(obuf0, acc.at[sdidx0], o0, add=True)
        pltpu.async_copy(obuf1, acc.at[sdidx1], o1, add=True)
        g_issue(0)
        plsc.subcore_barrier()

        bs = jnp.max(bvec[...])

        def c_main(p):
            sr, dr, _, _ = rows[p]
            ob, osem, sd = obufs[p]
            for g in range(B // LANES):
                erow = lane + (g * LANES)

                @pl.loop(0, PW, init_carry=jnp.zeros((LANES,), jnp.float32),
                         unroll=8)
                def dots(wi, acc16):
                    colv = jnp.full((LANES,), wi, dtype=jnp.int32)
                    sw = plsc.bitcast(plsc.load_gather(sr, [erow, colv]),
                                      jnp.bfloat16)
                    dw = plsc.bitcast(plsc.load_gather(dr, [erow, colv]),
                                      jnp.bfloat16)
                    lo, hi = plsc.unpack(sw * dw,
                                         format=plsc.PackFormat.INTERLEAVED,
                                         preferred_element_type=jnp.float32)
                    return acc16 + lo + hi

                cols = jnp.full((LANES,), PW, dtype=jnp.int32)
                norms16 = plsc.load_gather(sr, [erow, cols])
                wv16 = jnp.exp(bs * dots)
                sv16 = wv16 * norms16
                for l in range(LANES):
                    e = g * LANES + l
                    sv = jnp.squeeze(lax.slice(sv16, (l,), (l + 1,)))
                    w = jnp.squeeze(lax.slice(wv16, (l,), (l + 1,)))
                    svv = jnp.full((LANES,), sv)
                    for k in range(4):
                        wrd = plsc.bitcast(sr[e, pl.ds(k * LANES, LANES)],
                                           jnp.bfloat16)
                        lo, hi = plsc.unpack(
                            wrd, format=plsc.PackFormat.INTERLEAVED,
                            preferred_element_type=jnp.float32)
                        ob[e, pl.ds(32 * k, LANES)] = lo * svv
                        ob[e, pl.ds(32 * k + LANES, LANES)] = hi * svv
                    ob[e, pl.ds(D, LANES)] = jnp.where(lane == 0, w, 0.0)
            pltpu.async_copy(ob, acc.at[sd], osem, add=True)

        @pl.loop(0, NCHUNK - 1, step=2)
        def _pair(ci):
            i_wait(1, ci + 1)
            g_issue(1)
            g_wait(0)
            c_pre(0)

            @pl.when(ci + 2 < NCHUNK)
            def _():
                i_issue(0, ci + 2)

            c_main(0)
            g_wait(1)
            c_pre(1)

            @pl.when(ci + 3 < NCHUNK)
            def _():
                i_issue(1, ci + 3)

            c_main(1)

            @pl.when(ci + 2 < NCHUNK)
            def _():
                i_wait(0, ci + 2)
                g_issue(0)

        g_wait(0)
        c_pre(0)
        c_main(0)
        pltpu.make_async_copy(obuf0, acc.at[sdidx0], o0).wait()
        pltpu.make_async_copy(obuf1, acc.at[sdidx1], o1).wait()
        plsc.subcore_barrier()
        pltpu.sync_copy(acc.at[pl.ds(row0, RPT)],
                        out.at[pl.ds(c * N + row0, RPT)])

    return sc


def kernel(x, edge_index, beta):
    N = x.shape[0]
    E = edge_index.shape[1]
    NW = 32
    table = pl.pallas_call(
        _norm_body,
        grid=(N // BN,),
        in_specs=[pl.BlockSpec((BN, D), lambda i: (i, 0))],
        out_specs=pl.BlockSpec((BN, TW), lambda i: (i, 0)),
        out_shape=jax.ShapeDtypeStruct((N, TW), jnp.float32),
    )(x)
    epw = E // NW
    epwp = -(-epw // B) * B
    pad = NW * epwp - E
    src = jnp.concatenate(
        [edge_index[0], jnp.zeros((pad,), jnp.int32)]).reshape(NW, epwp)
    dst = jnp.concatenate(
        [edge_index[1], jnp.full((pad,), N, jnp.int32)]).reshape(NW, epwp)
    beta16 = jnp.broadcast_to(beta.astype(jnp.float32), (LANES,))
    accflat = _make_sc(N, epwp)(table, src, dst, beta16)
    nb = N // BN
    out = pl.pallas_call(
        _combine_body,
        grid=(nb,),
        in_specs=[pl.BlockSpec((BN, DP), lambda i: (i, 0)),
                  pl.BlockSpec((BN, DP), lambda i: (i + nb, 0))],
        out_specs=pl.BlockSpec((BN, D), lambda i: (i, 0)),
        out_shape=jax.ShapeDtypeStruct((N, D), jnp.float32),
    )(accflat, accflat)
    return out


# speedup vs baseline: 1.2660x; 1.2660x over previous
import functools

import jax
import jax.numpy as jnp
from jax import lax
from jax.experimental import pallas as pl
from jax.experimental.pallas import tpu as pltpu
from jax.experimental.pallas import tpu_sc as plsc

D = 128
TAIL = 16
DP = D + TAIL
PW = D // 2
TW = PW + TAIL
LANES = 16
BN = 1000
B = 48


def _norm_body(x_ref, out_ref):
    x = x_ref[...]
    nrm = jnp.sqrt(jnp.sum(x * x, axis=1, keepdims=True))
    xn = x / (nrm + 1e-12)
    u = jax.lax.bitcast_convert_type(xn.astype(jnp.bfloat16),
                                     jnp.uint16).astype(jnp.uint32)
    groups = []
    for k in range(4):
        lo = u[:, 32 * k:32 * k + 16]
        hi = u[:, 32 * k + 16:32 * k + 32]
        groups.append((hi << 16) | lo)
    packed = jax.lax.bitcast_convert_type(
        jnp.concatenate(groups, axis=1), jnp.float32)
    tail = jnp.broadcast_to(nrm, (x.shape[0], TAIL))
    out_ref[...] = jnp.concatenate([packed, tail], axis=1)


def _combine_body(a0_ref, a1_ref, out_ref):
    s = a0_ref[...] + a1_ref[...]
    out_ref[...] = s[:, :D] / (s[:, D:D + 1] + 1e-16)


@functools.lru_cache(maxsize=None)
def _make_sc(N, EPWP):
    info = plsc.get_sparse_core_info()
    NC, NS = info.num_cores, info.num_subcores
    NCHUNK = EPWP // B
    RPT = N // NS
    NZ = RPT // B
    REM = RPT - NZ * B
    mesh = plsc.VectorSubcoreMesh(core_axis_name="c", subcore_axis_name="s")

    @functools.partial(
        pl.kernel,
        out_type=jax.ShapeDtypeStruct((NC * N, DP), jnp.float32),
        mesh=mesh,
        compiler_params=pltpu.CompilerParams(use_tc_tiling_on_sc=False,
                                             needs_layout_passes=False),
        scratch_types=[
            pltpu.VMEM((B,), jnp.int32),
            pltpu.VMEM((B,), jnp.int32),
            pltpu.VMEM((B,), jnp.int32),
            pltpu.VMEM((B,), jnp.int32),
            pltpu.VMEM((B, TW), jnp.float32),
            pltpu.VMEM((B, TW), jnp.float32),
            pltpu.VMEM((B, TW), jnp.float32),
            pltpu.VMEM((B, TW), jnp.float32),
            pltpu.VMEM((B, DP), jnp.float32),
            pltpu.VMEM((B, DP), jnp.float32),
            pltpu.VMEM((B,), jnp.int32),
            pltpu.VMEM((B,), jnp.int32),
            pltpu.VMEM((LANES,), jnp.float32),
            pltpu.VMEM_SHARED((N + 1, DP), jnp.float32),
        ] + [pltpu.SemaphoreType.DMA] * 8,
    )
    def sc(table, srcs2, dsts2, beta16, out,
           sidx0, didx0, sidx1, didx1, sr0, dr0, sr1, dr1,
           obuf0, obuf1, sdidx0, sdidx1, bvec, acc,
           gs0, gd0, gs1, gd1, o0, o1, i0, i1):
        c = lax.axis_index("c")
        s = lax.axis_index("s")
        wid = c * NS + s
        zero16 = jnp.zeros((LANES,), jnp.float32)
        idxs = ((sidx0, didx0, i0), (sidx1, didx1, i1))
        rows = ((sr0, dr0, gs0, gd0), (sr1, dr1, gs1, gd1))
        obufs = ((obuf0, o0, sdidx0), (obuf1, o1, sdidx1))

        @pl.loop(0, B)
        def _zero_obuf(r):
            for k in range(DP // LANES):
                obuf0[r, pl.ds(k * LANES, LANES)] = zero16
                obuf1[r, pl.ds(k * LANES, LANES)] = zero16

        lane = lax.iota(jnp.int32, LANES)

        row0 = s * RPT
        for j in range(NZ):
            pltpu.sync_copy(obuf0, acc.at[pl.ds(row0 + j * B, B)])
        if REM:
            pltpu.sync_copy(obuf0.at[pl.ds(0, REM)],
                            acc.at[pl.ds(row0 + NZ * B, REM)])
        pltpu.sync_copy(beta16, bvec)

        def i_issue(p, ci):
            si, di, isem = idxs[p]
            pltpu.async_copy(srcs2.at[wid, pl.ds(ci * B, B)], si, isem)
            pltpu.async_copy(dsts2.at[wid, pl.ds(ci * B, B)], di, isem)

        def i_wait(p, ci):
            si, di, isem = idxs[p]
            pltpu.make_async_copy(srcs2.at[wid, pl.ds(ci * B, B)],
                                  si, isem).wait()
            pltpu.make_async_copy(dsts2.at[wid, pl.ds(ci * B, B)],
                                  di, isem).wait()

        def g_issue(p):
            si, di, _ = idxs[p]
            sr, dr, ss, sd = rows[p]
            pltpu.async_copy(table.at[si], sr, ss)
            pltpu.async_copy(table.at[di], dr, sd)

        def g_wait(p):
            si, di, _ = idxs[p]
            sr, dr, ss, sd = rows[p]
            pltpu.make_async_copy(table.at[si], sr, ss).wait()
            pltpu.make_async_copy(table.at[di], dr, sd).wait()

        def c_pre(p):
            _, di, _ = idxs[p]
            ob, osem, sd = obufs[p]
            pltpu.make_async_copy(ob, acc.at[sd], osem).wait()
            for t in range(B // LANES):
                sd[pl.ds(t * LANES, LANES)] = di[pl.ds(t * LANES, LANES)]

        pltpu.sync_copy(srcs2.at[wid, pl.ds(0, B)], sidx0)
        pltpu.sync_copy(dsts2.at[wid, pl.ds(0, B)], didx0)
        i_issue(1, 1)
        for t in range(B // LANES):
            sdidx0[pl.ds(t * LANES, LANES)] = didx0[pl.ds(t * LANES, LANES)]
            sdidx1[pl.ds(t * LANES, LANES)] = didx0[pl.ds(t * LANES, LANES)]
        pltpu.async_copy(obuf0, acc.at[sdidx0], o0, add=True)
        pltpu.async_copy(obuf1, acc.at[sdidx1], o1, add=True)
        g_issue(0)
        plsc.subcore_barrier()

        bs = jnp.max(bvec[...])

        def c_main(p):
            sr, dr, _, _ = rows[p]
            ob, osem, sd = obufs[p]
            for g in range(B // LANES):
                erow = lane + (g * LANES)

                @pl.loop(0, PW, init_carry=jnp.zeros((LANES,), jnp.float32),
                         unroll=8)
                def dots(wi, acc16):
                    colv = (lane + wi) & (PW - 1)
                    sw = plsc.bitcast(plsc.load_gather(sr, [erow, colv]),
                                      jnp.bfloat16)
                    dw = plsc.bitcast(plsc.load_gather(dr, [erow, colv]),
                                      jnp.bfloat16)
                    lo, hi = plsc.unpack(sw * dw,
                                         format=plsc.PackFormat.INTERLEAVED,
                                         preferred_element_type=jnp.float32)
                    return acc16 + lo + hi

                norms16 = plsc.load_gather(sr, [erow, lane + PW])
                wv16 = jnp.exp(bs * dots)
                sv16 = wv16 * norms16
                for l in range(LANES):
                    e = g * LANES + l
                    sv = jnp.squeeze(lax.slice(sv16, (l,), (l + 1,)))
                    w = jnp.squeeze(lax.slice(wv16, (l,), (l + 1,)))
                    svv = jnp.full((LANES,), sv)
                    for k in range(4):
                        wrd = plsc.bitcast(sr[e, pl.ds(k * LANES, LANES)],
                                           jnp.bfloat16)
                        lo, hi = plsc.unpack(
                            wrd, format=plsc.PackFormat.INTERLEAVED,
                            preferred_element_type=jnp.float32)
                        ob[e, pl.ds(32 * k, LANES)] = lo * svv
                        ob[e, pl.ds(32 * k + LANES, LANES)] = hi * svv
                    ob[e, pl.ds(D, LANES)] = jnp.where(lane == 0, w, 0.0)
            pltpu.async_copy(ob, acc.at[sd], osem, add=True)

        @pl.loop(0, NCHUNK - 1, step=2)
        def _pair(ci):
            i_wait(1, ci + 1)
            g_issue(1)
            g_wait(0)
            c_pre(0)

            @pl.when(ci + 2 < NCHUNK)
            def _():
                i_issue(0, ci + 2)

            c_main(0)
            g_wait(1)
            c_pre(1)

            @pl.when(ci + 3 < NCHUNK)
            def _():
                i_issue(1, ci + 3)

            c_main(1)

            @pl.when(ci + 2 < NCHUNK)
            def _():
                i_wait(0, ci + 2)
                g_issue(0)

        g_wait(0)
        c_pre(0)
        c_main(0)
        pltpu.make_async_copy(obuf0, acc.at[sdidx0], o0).wait()
        pltpu.make_async_copy(obuf1, acc.at[sdidx1], o1).wait()
        plsc.subcore_barrier()
        pltpu.sync_copy(acc.at[pl.ds(row0, RPT)],
                        out.at[pl.ds(c * N + row0, RPT)])

    return sc


def kernel(x, edge_index, beta):
    N = x.shape[0]
    E = edge_index.shape[1]
    NW = 32
    table = pl.pallas_call(
        _norm_body,
        grid=(N // BN,),
        in_specs=[pl.BlockSpec((BN, D), lambda i: (i, 0))],
        out_specs=pl.BlockSpec((BN, TW), lambda i: (i, 0)),
        out_shape=jax.ShapeDtypeStruct((N, TW), jnp.float32),
    )(x)
    epw = E // NW
    epwp = -(-epw // B) * B
    pad = NW * epwp - E
    src = jnp.concatenate(
        [edge_index[0], jnp.zeros((pad,), jnp.int32)]).reshape(NW, epwp)
    dst = jnp.concatenate(
        [edge_index[1], jnp.full((pad,), N, jnp.int32)]).reshape(NW, epwp)
    beta16 = jnp.broadcast_to(beta.astype(jnp.float32), (LANES,))
    accflat = _make_sc(N, epwp)(table, src, dst, beta16)
    nb = N // BN
    out = pl.pallas_call(
        _combine_body,
        grid=(nb,),
        in_specs=[pl.BlockSpec((BN, DP), lambda i: (i, 0)),
                  pl.BlockSpec((BN, DP), lambda i: (i + nb, 0))],
        out_specs=pl.BlockSpec((BN, D), lambda i: (i, 0)),
        out_shape=jax.ShapeDtypeStruct((N, D), jnp.float32),
    )(accflat, accflat)
    return out
